# baseline (speedup 1.0000x reference)
import functools

import jax
import jax.numpy as jnp
from jax import lax
from jax.experimental import pallas as pl
from jax.experimental.pallas import tpu as pltpu
from jax.experimental.pallas import tpu_sc as plsc

N = 10000
E = 320000
D = 128
DH = D // 2

NC = 2
NS = 16
CHUNK = 128
NBUF = 3
KAH = 2
NCH = -(-E // (NS * CHUNK * NBUF)) * NBUF
E_PAD = NCH * CHUNK * NS
NCH1 = NCH + KAH

N_PAD = 10240
RPT = N_PAD // NS
RPX = N // NS
PBITS = 14
PMASK = (1 << PBITS) - 1


def _make_sc_agg():
    mesh = plsc.VectorSubcoreMesh(core_axis_name="c", subcore_axis_name="s")

    @functools.partial(
        pl.kernel,
        mesh=mesh,
        out_type=jax.ShapeDtypeStruct((NC, N_PAD, DH), jnp.float32),
        scratch_types=[
            pltpu.VMEM((NCH1, CHUNK), jnp.int32),
            pltpu.VMEM((NBUF, 2, CHUNK), jnp.int32),
            pltpu.VMEM((NBUF, CHUNK, DH), jnp.float32),
            pltpu.VMEM_SHARED((N, DH), jnp.float32),
            pltpu.VMEM_SHARED((N_PAD, DH), jnp.float32),
            [pltpu.SemaphoreType.DMA] * NBUF,
            [pltpu.SemaphoreType.DMA] * NBUF,
        ],
        compiler_params=pltpu.CompilerParams(use_tc_tiling_on_sc=False),
    )
    def sc_agg(xh_hbm, ids_hbm, out_hbm,
               pk_v, unp, bufs, x_sh, acc_sh, gsems, ssems):
        c = lax.axis_index("c")
        s = lax.axis_index("s")

        def zbody(i, carry):
            r = i // (DH // 16)
            k = i - r * (DH // 16)
            bufs[0, r, pl.ds(k * 16, 16)] = jnp.zeros((16,), jnp.float32)
            return carry

        lax.fori_loop(0, CHUNK * (DH // 16), zbody, 0)
        for r5 in range(RPT // CHUNK):
            pltpu.sync_copy(bufs.at[0],
                            acc_sh.at[pl.ds(s * RPT + r5 * CHUNK, CHUNK)])
        pltpu.sync_copy(xh_hbm.at[pl.ds(c * N + s * RPX, RPX)],
                        x_sh.at[pl.ds(s * RPX, RPX)])
        pltpu.sync_copy(ids_hbm.at[s], pk_v)
        plsc.subcore_barrier()

        def unpack(j, b):
            for k in range(CHUNK // 16):
                p = pk_v[j, pl.ds(k * 16, 16)]
                unp[b, 0, pl.ds(k * 16, 16)] = lax.shift_right_logical(
                    p, PBITS)
                unp[b, 1, pl.ds(k * 16, 16)] = lax.bitwise_and(
                    p, jnp.int32(PMASK))

        def gather(j, b):
            pltpu.async_copy(x_sh.at[unp.at[b, 0]], bufs.at[b], gsems[b])

        def gather_wait(j, b):
            pltpu.make_async_copy(x_sh.at[unp.at[b, 0]], bufs.at[b],
                                  gsems[b]).wait()

        def scatter(j, b):
            pltpu.async_copy(bufs.at[b], acc_sh.at[unp.at[b, 1]],
                             ssems[b], add=True)

        def scatter_wait(j, b):
            pltpu.make_async_copy(bufs.at[b], acc_sh.at[unp.at[b, 1]],
                                  ssems[b]).wait()

        for j in range(KAH):
            unpack(j, j)
            gather(j, j)
        for j in range(NBUF - KAH):
            b, bg = j % NBUF, (j + KAH) % NBUF
            unpack(j + KAH, bg)
            gather(j + KAH, bg)
            gather_wait(j, b)
            scatter(j, b)
        for j in range(NBUF - KAH, NBUF):
            b, bg = j % NBUF, (j + KAH) % NBUF
            scatter_wait(j + KAH - NBUF, bg)
            unpack(j + KAH, bg)
            gather(j + KAH, bg)
            gather_wait(j, b)
            scatter(j, b)

        def body(t, carry):
            for r in range(NBUF):
                j = t * NBUF + r
                bg = (r + KAH) % NBUF
                scatter_wait(j + KAH - NBUF, bg)
                unpack(j + KAH, bg)
                gather(j + KAH, bg)
                gather_wait(j, r)
                scatter(j, r)
            return carry

        lax.fori_loop(1, NCH // NBUF, body, 0)
        for j in range(NCH, NCH + KAH):
            gather_wait(j, j % NBUF)
        for j in range(NCH + KAH - NBUF, NCH):
            scatter_wait(j, j % NBUF)
        plsc.subcore_barrier()

        pltpu.sync_copy(acc_sh.at[pl.ds(s * RPT, RPT)],
                        out_hbm.at[c].at[pl.ds(s * RPT, RPT)])

    return sc_agg


_sc_agg = _make_sc_agg()


def _epilogue_body(p_ref, deg_ref, w_ref, b_ref, o_ref):
    ssum = jnp.concatenate([p_ref[0], p_ref[1]], axis=1)
    ssum = ssum / deg_ref[...]
    o_ref[...] = lax.dot_general(
        ssum, w_ref[...], (((1,), (1,)), ((), ())),
        preferred_element_type=jnp.float32) + b_ref[...]


_BM = 2048

_epilogue = pl.pallas_call(
    _epilogue_body,
    grid=(N_PAD // _BM,),
    in_specs=[
        pl.BlockSpec((NC, _BM, DH), lambda i: (0, i, 0)),
        pl.BlockSpec((_BM, 1), lambda i: (i, 0)),
        pl.BlockSpec((D, D), lambda i: (0, 0)),
        pl.BlockSpec((1, D), lambda i: (0, 0)),
    ],
    out_specs=pl.BlockSpec((_BM, D), lambda i: (i, 0)),
    out_shape=jax.ShapeDtypeStruct((N_PAD, D), jnp.float32),
)


def kernel(x, edge_index, node_degrees, W, b):
    src = edge_index[0]
    dst = edge_index[1]
    pad = E_PAD - E
    packed = jnp.concatenate(
        [lax.shift_left(src, PBITS) | dst,
         jnp.full((pad,), N, jnp.int32)])
    ids = jnp.concatenate(
        [packed.reshape(NS, NCH, CHUNK),
         jnp.full((NS, KAH, CHUNK), N, jnp.int32)], axis=1)
    xh = jnp.concatenate([x[:, :DH], x[:, DH:]], axis=0)

    parts = _sc_agg(xh, ids)

    deg_p = jnp.concatenate(
        [node_degrees, jnp.ones((N_PAD - N,), jnp.float32)]).reshape(N_PAD, 1)
    out = _epilogue(parts, deg_p, W, b.reshape(1, D))
    return out[:N]


# speedup vs baseline: 1.1635x; 1.1635x over previous
import functools

import jax
import jax.numpy as jnp
from jax import lax
from jax.experimental import pallas as pl
from jax.experimental.pallas import tpu as pltpu
from jax.experimental.pallas import tpu_sc as plsc

N = 10000
E = 320000
D = 128
DH = D // 2

NC = 2
NS = 16
CHUNK = 128
NBUF = 3
KAH = 1
NCH = -(-E // (NS * CHUNK * NBUF)) * NBUF
E_PAD = NCH * CHUNK * NS
NCH1 = NCH + KAH

N_PAD = 10240
RPT = N_PAD // NS
RPX = N // NS
PBITS = 14
PMASK = (1 << PBITS) - 1


def _make_sc_agg():
    mesh = plsc.VectorSubcoreMesh(core_axis_name="c", subcore_axis_name="s")

    @functools.partial(
        pl.kernel,
        mesh=mesh,
        out_type=jax.ShapeDtypeStruct((NC, N_PAD, DH), jnp.float32),
        scratch_types=[
            pltpu.VMEM((NCH1, CHUNK), jnp.int32),
            pltpu.VMEM((NBUF, 1, CHUNK), jnp.int32),
            pltpu.VMEM((NBUF, 2, CHUNK // 2), jnp.int32),
            pltpu.VMEM((NBUF, CHUNK, DH), jnp.float32),
            pltpu.VMEM_SHARED((N, DH), jnp.float32),
            pltpu.VMEM_SHARED((N_PAD, DH), jnp.float32),
            [pltpu.SemaphoreType.DMA] * NBUF,
            [pltpu.SemaphoreType.DMA] * (2 * NBUF),
        ],
        compiler_params=pltpu.CompilerParams(use_tc_tiling_on_sc=False),
    )
    def sc_agg(xh_hbm, ids_hbm, out_hbm,
               pk_v, unp, unpd, bufs, x_sh, acc_sh, gsems, ssems):
        c = lax.axis_index("c")
        s = lax.axis_index("s")

        def zbody(i, carry):
            r = i // (DH // 16)
            k = i - r * (DH // 16)
            bufs[0, r, pl.ds(k * 16, 16)] = jnp.zeros((16,), jnp.float32)
            return carry

        lax.fori_loop(0, CHUNK * (DH // 16), zbody, 0)
        for r5 in range(RPT // CHUNK):
            pltpu.sync_copy(bufs.at[0],
                            acc_sh.at[pl.ds(s * RPT + r5 * CHUNK, CHUNK)])
        pltpu.sync_copy(xh_hbm.at[pl.ds(c * N + s * RPX, RPX)],
                        x_sh.at[pl.ds(s * RPX, RPX)])
        pltpu.sync_copy(ids_hbm.at[s], pk_v)
        plsc.subcore_barrier()

        def unpack(j, b):
            for k in range(CHUNK // 16):
                p = pk_v[j, pl.ds(k * 16, 16)]
                unp[b, 0, pl.ds(k * 16, 16)] = lax.shift_right_logical(
                    p, PBITS)
                h, kk = divmod(k, CHUNK // 32)
                unpd[b, h, pl.ds(kk * 16, 16)] = lax.bitwise_and(
                    p, jnp.int32(PMASK))

        def gather(j, b):
            pltpu.async_copy(x_sh.at[unp.at[b, 0]], bufs.at[b], gsems[b])

        def gather_wait(j, b):
            pltpu.make_async_copy(x_sh.at[unp.at[b, 0]], bufs.at[b],
                                  gsems[b]).wait()

        def scatter(j, b):
            for h in range(2):
                pltpu.async_copy(bufs.at[b, pl.ds(h * (CHUNK // 2), CHUNK // 2)],
                                 acc_sh.at[unpd.at[b, h]],
                                 ssems[2 * b + h], add=True)

        def scatter_wait(j, b):
            for h in range(2):
                pltpu.make_async_copy(
                    bufs.at[b, pl.ds(h * (CHUNK // 2), CHUNK // 2)],
                    acc_sh.at[unpd.at[b, h]],
                    ssems[2 * b + h]).wait()

        for j in range(KAH):
            unpack(j, j)
            gather(j, j)
        for j in range(NBUF - KAH):
            b, bg = j % NBUF, (j + KAH) % NBUF
            unpack(j + KAH, bg)
            gather(j + KAH, bg)
            gather_wait(j, b)
            scatter(j, b)
        for j in range(NBUF - KAH, NBUF):
            b, bg = j % NBUF, (j + KAH) % NBUF
            scatter_wait(j + KAH - NBUF, bg)
            unpack(j + KAH, bg)
            gather(j + KAH, bg)
            gather_wait(j, b)
            scatter(j, b)

        def body(t, carry):
            for r in range(NBUF):
                j = t * NBUF + r
                bg = (r + KAH) % NBUF
                scatter_wait(j + KAH - NBUF, bg)
                unpack(j + KAH, bg)
                gather(j + KAH, bg)
                gather_wait(j, r)
                scatter(j, r)
            return carry

        lax.fori_loop(1, NCH // NBUF, body, 0)
        for j in range(NCH, NCH + KAH):
            gather_wait(j, j % NBUF)
        for j in range(NCH + KAH - NBUF, NCH):
            scatter_wait(j, j % NBUF)
        plsc.subcore_barrier()

        pltpu.sync_copy(acc_sh.at[pl.ds(s * RPT, RPT)],
                        out_hbm.at[c].at[pl.ds(s * RPT, RPT)])

    return sc_agg


_sc_agg = _make_sc_agg()


def _epilogue_body(p_ref, deg_ref, w_ref, b_ref, o_ref):
    ssum = jnp.concatenate([p_ref[0], p_ref[1]], axis=1)
    ssum = ssum / deg_ref[...]
    o_ref[...] = lax.dot_general(
        ssum, w_ref[...], (((1,), (1,)), ((), ())),
        preferred_element_type=jnp.float32) + b_ref[...]


_BM = 2048

_epilogue = pl.pallas_call(
    _epilogue_body,
    grid=(N_PAD // _BM,),
    in_specs=[
        pl.BlockSpec((NC, _BM, DH), lambda i: (0, i, 0)),
        pl.BlockSpec((_BM, 1), lambda i: (i, 0)),
        pl.BlockSpec((D, D), lambda i: (0, 0)),
        pl.BlockSpec((1, D), lambda i: (0, 0)),
    ],
    out_specs=pl.BlockSpec((_BM, D), lambda i: (i, 0)),
    out_shape=jax.ShapeDtypeStruct((N_PAD, D), jnp.float32),
)


def kernel(x, edge_index, node_degrees, W, b):
    src = edge_index[0]
    dst = edge_index[1]
    pad = E_PAD - E
    packed = jnp.concatenate(
        [lax.shift_left(src, PBITS) | dst,
         jnp.full((pad,), N, jnp.int32)])
    ids = jnp.concatenate(
        [packed.reshape(NS, NCH, CHUNK),
         jnp.full((NS, KAH, CHUNK), N, jnp.int32)], axis=1)
    xh = jnp.concatenate([x[:, :DH], x[:, DH:]], axis=0)

    parts = _sc_agg(xh, ids)

    deg_p = jnp.concatenate(
        [node_degrees, jnp.ones((N_PAD - N,), jnp.float32)]).reshape(N_PAD, 1)
    out = _epilogue(parts, deg_p, W, b.reshape(1, D))
    return out[:N]


# speedup vs baseline: 1.3365x; 1.1488x over previous
import functools

import jax
import jax.numpy as jnp
from jax import lax
from jax.experimental import pallas as pl
from jax.experimental.pallas import tpu as pltpu
from jax.experimental.pallas import tpu_sc as plsc

N = 10000
E = 320000
D = 128
DH = D // 2

NC = 2
NS = 16
CHUNK = 128
NBUF = 3
KAH = 1
NCH = -(-E // (NS * CHUNK * NBUF)) * NBUF
E_PAD = NCH * CHUNK * NS

N_PAD = 10240
RPT = N_PAD // NS
RPX = N // NS
PBITS = 14
PMASK = (1 << PBITS) - 1


def _make_sc_agg():
    mesh = plsc.VectorSubcoreMesh(core_axis_name="c", subcore_axis_name="s")

    @functools.partial(
        pl.kernel,
        mesh=mesh,
        out_type=jax.ShapeDtypeStruct((NC, N_PAD, DH), jnp.float32),
        scratch_types=[
            pltpu.VMEM((NCH, CHUNK), jnp.int32),
            pltpu.VMEM((NBUF, 1, CHUNK), jnp.int32),
            pltpu.VMEM((NBUF, 2, CHUNK // 2), jnp.int32),
            pltpu.VMEM((NBUF, CHUNK, DH), jnp.float32),
            pltpu.VMEM_SHARED((N, DH), jnp.float32),
            pltpu.VMEM_SHARED((N_PAD, DH), jnp.float32),
            [pltpu.SemaphoreType.DMA] * NBUF,
            [pltpu.SemaphoreType.DMA] * (2 * NBUF),
        ],
        compiler_params=pltpu.CompilerParams(use_tc_tiling_on_sc=False),
    )
    def sc_agg(x_hbm, ids_hbm, out_hbm,
               pk_v, unp, unpd, bufs, x_sh, acc_sh, gsems, ssems):
        c = lax.axis_index("c")
        s = lax.axis_index("s")

        def zbody(i, carry):
            r = i // (DH // 16)
            k = i - r * (DH // 16)
            bufs[0, r, pl.ds(k * 16, 16)] = jnp.zeros((16,), jnp.float32)
            return carry

        lax.fori_loop(0, CHUNK * (DH // 16), zbody, 0)
        for r5 in range(RPT // CHUNK):
            pltpu.sync_copy(bufs.at[0],
                            acc_sh.at[pl.ds(s * RPT + r5 * CHUNK, CHUNK)])
        pltpu.sync_copy(x_hbm.at[pl.ds(s * RPX, RPX), pl.ds(c * DH, DH)],
                        x_sh.at[pl.ds(s * RPX, RPX)])
        pltpu.sync_copy(ids_hbm.at[s], pk_v)
        plsc.subcore_barrier()

        def unpack(j, b):
            jc = jnp.minimum(jnp.int32(j), jnp.int32(NCH - 1))
            for k in range(CHUNK // 16):
                p = pk_v[jc, pl.ds(k * 16, 16)]
                unp[b, 0, pl.ds(k * 16, 16)] = lax.shift_right_logical(
                    p, PBITS)
                h, kk = divmod(k, CHUNK // 32)
                unpd[b, h, pl.ds(kk * 16, 16)] = lax.bitwise_and(
                    p, jnp.int32(PMASK))

        def gather(j, b):
            pltpu.async_copy(x_sh.at[unp.at[b, 0]], bufs.at[b], gsems[b])

        def gather_wait(j, b):
            pltpu.make_async_copy(x_sh.at[unp.at[b, 0]], bufs.at[b],
                                  gsems[b]).wait()

        def scatter(j, b):
            for h in range(2):
                pltpu.async_copy(bufs.at[b, pl.ds(h * (CHUNK // 2), CHUNK // 2)],
                                 acc_sh.at[unpd.at[b, h]],
                                 ssems[2 * b + h], add=True)

        def scatter_wait(j, b):
            for h in range(2):
                pltpu.make_async_copy(
                    bufs.at[b, pl.ds(h * (CHUNK // 2), CHUNK // 2)],
                    acc_sh.at[unpd.at[b, h]],
                    ssems[2 * b + h]).wait()

        for j in range(KAH):
            unpack(j, j)
            gather(j, j)
        for j in range(NBUF - KAH):
            b, bg = j % NBUF, (j + KAH) % NBUF
            unpack(j + KAH, bg)
            gather(j + KAH, bg)
            gather_wait(j, b)
            scatter(j, b)
        for j in range(NBUF - KAH, NBUF):
            b, bg = j % NBUF, (j + KAH) % NBUF
            scatter_wait(j + KAH - NBUF, bg)
            unpack(j + KAH, bg)
            gather(j + KAH, bg)
            gather_wait(j, b)
            scatter(j, b)

        def body(t, carry):
            for r in range(NBUF):
                j = t * NBUF + r
                bg = (r + KAH) % NBUF
                scatter_wait(j + KAH - NBUF, bg)
                unpack(j + KAH, bg)
                gather(j + KAH, bg)
                gather_wait(j, r)
                scatter(j, r)
            return carry

        lax.fori_loop(1, NCH // NBUF, body, 0)
        for j in range(NCH, NCH + KAH):
            gather_wait(j, j % NBUF)
        for j in range(NCH + KAH - NBUF, NCH):
            scatter_wait(j, j % NBUF)
        plsc.subcore_barrier()

        pltpu.sync_copy(acc_sh.at[pl.ds(s * RPT, RPT)],
                        out_hbm.at[c].at[pl.ds(s * RPT, RPT)])

    return sc_agg


_sc_agg = _make_sc_agg()


def _epilogue_body(p_ref, deg_ref, w_ref, b_ref, o_ref):
    ssum = jnp.concatenate([p_ref[0], p_ref[1]], axis=1)
    ssum = ssum / deg_ref[...]
    o_ref[...] = lax.dot_general(
        ssum, w_ref[...], (((1,), (1,)), ((), ())),
        preferred_element_type=jnp.float32) + b_ref[...]


_BM = 2000

_epilogue = pl.pallas_call(
    _epilogue_body,
    grid=(N // _BM,),
    in_specs=[
        pl.BlockSpec((NC, _BM, DH), lambda i: (0, i, 0)),
        pl.BlockSpec((_BM, 1), lambda i: (i, 0)),
        pl.BlockSpec((D, D), lambda i: (0, 0)),
        pl.BlockSpec((1, D), lambda i: (0, 0)),
    ],
    out_specs=pl.BlockSpec((_BM, D), lambda i: (i, 0)),
    out_shape=jax.ShapeDtypeStruct((N, D), jnp.float32),
)


def kernel(x, edge_index, node_degrees, W, b):
    src = edge_index[0]
    dst = edge_index[1]
    pad = E_PAD - E
    packed = jnp.concatenate(
        [lax.shift_left(src, PBITS) | dst,
         jnp.full((pad,), N, jnp.int32)])
    ids = packed.reshape(NS, NCH, CHUNK)

    parts = _sc_agg(x, ids)

    return _epilogue(parts, node_degrees.reshape(N, 1), W, b.reshape(1, D))


# speedup vs baseline: 1.3644x; 1.0208x over previous
import functools

import jax
import jax.numpy as jnp
from jax import lax
from jax.experimental import pallas as pl
from jax.experimental.pallas import tpu as pltpu
from jax.experimental.pallas import tpu_sc as plsc

N = 10000
E = 320000
D = 128
DH = D // 2

NC = 2
NS = 16
CHUNK = 128
NBUF = 3
KAH = 1
NCH = -(-E // (NS * CHUNK * NBUF)) * NBUF
E_PAD = NCH * CHUNK * NS

N_PAD = 10240
RPT = N_PAD // NS
RPX = N // NS
PBITS = 14
PMASK = (1 << PBITS) - 1


def _make_sc_agg():
    mesh = plsc.VectorSubcoreMesh(core_axis_name="c", subcore_axis_name="s")

    @functools.partial(
        pl.kernel,
        mesh=mesh,
        out_type=jax.ShapeDtypeStruct((NC, N_PAD, DH), jnp.float32),
        scratch_types=[
            pltpu.VMEM((NCH, CHUNK), jnp.int32),
            pltpu.VMEM((NBUF, 1, CHUNK), jnp.int32),
            pltpu.VMEM((NBUF, 2, CHUNK // 2), jnp.int32),
            pltpu.VMEM((NBUF, CHUNK, DH), jnp.float32),
            pltpu.VMEM_SHARED((N, DH), jnp.float32),
            pltpu.VMEM_SHARED((N_PAD, DH), jnp.float32),
            [pltpu.SemaphoreType.DMA] * NBUF,
            [pltpu.SemaphoreType.DMA] * (2 * NBUF),
        ],
        compiler_params=pltpu.CompilerParams(use_tc_tiling_on_sc=False),
    )
    def sc_agg(x_hbm, ids_hbm, out_hbm,
               pk_v, unp, unpd, bufs, x_sh, acc_sh, gsems, ssems):
        c = lax.axis_index("c")
        s = lax.axis_index("s")

        def zbody(i, carry):
            r = i // (DH // 16)
            k = i - r * (DH // 16)
            bufs[0, r, pl.ds(k * 16, 16)] = jnp.zeros((16,), jnp.float32)
            return carry

        lax.fori_loop(0, CHUNK * (DH // 16), zbody, 0)
        for r5 in range(RPT // CHUNK):
            pltpu.sync_copy(bufs.at[0],
                            acc_sh.at[pl.ds(s * RPT + r5 * CHUNK, CHUNK)])
        pltpu.sync_copy(x_hbm.at[pl.ds(s * RPX, RPX), pl.ds(c * DH, DH)],
                        x_sh.at[pl.ds(s * RPX, RPX)])
        pltpu.sync_copy(ids_hbm.at[s], pk_v)
        plsc.subcore_barrier()

        def unpack(j, b):
            jc = jnp.minimum(jnp.int32(j), jnp.int32(NCH - 1))
            for k in range(CHUNK // 16):
                p = pk_v[jc, pl.ds(k * 16, 16)]
                unp[b, 0, pl.ds(k * 16, 16)] = lax.shift_right_logical(
                    p, PBITS)
                h, kk = divmod(k, CHUNK // 32)
                unpd[b, h, pl.ds(kk * 16, 16)] = lax.bitwise_and(
                    p, jnp.int32(PMASK))

        def gather(j, b):
            pltpu.async_copy(x_sh.at[unp.at[b, 0]], bufs.at[b], gsems[b])

        def gather_wait(j, b):
            pltpu.make_async_copy(x_sh.at[unp.at[b, 0]], bufs.at[b],
                                  gsems[b]).wait()

        def scatter(j, b):
            for h in range(2):
                pltpu.async_copy(bufs.at[b, pl.ds(h * (CHUNK // 2), CHUNK // 2)],
                                 acc_sh.at[unpd.at[b, h]],
                                 ssems[2 * b + h], add=True)

        def scatter_wait(j, b):
            for h in range(2):
                pltpu.make_async_copy(
                    bufs.at[b, pl.ds(h * (CHUNK // 2), CHUNK // 2)],
                    acc_sh.at[unpd.at[b, h]],
                    ssems[2 * b + h]).wait()

        for j in range(KAH):
            unpack(j, j)
            gather(j, j)
        for j in range(NBUF - KAH):
            b, bg = j % NBUF, (j + KAH) % NBUF
            unpack(j + KAH, bg)
            gather(j + KAH, bg)
            gather_wait(j, b)
            scatter(j, b)
        for j in range(NBUF - KAH, NBUF):
            b, bg = j % NBUF, (j + KAH) % NBUF
            scatter_wait(j + KAH - NBUF, bg)
            unpack(j + KAH, bg)
            gather(j + KAH, bg)
            gather_wait(j, b)
            scatter(j, b)

        def body(t, carry):
            for r in range(NBUF):
                j = t * NBUF + r
                bg = (r + KAH) % NBUF
                scatter_wait(j + KAH - NBUF, bg)
                unpack(j + KAH, bg)
                gather(j + KAH, bg)
                gather_wait(j, r)
                scatter(j, r)
            return carry

        lax.fori_loop(1, NCH // NBUF, body, 0)
        for j in range(NCH, NCH + KAH):
            gather_wait(j, j % NBUF)
        for j in range(NCH + KAH - NBUF, NCH):
            scatter_wait(j, j % NBUF)
        plsc.subcore_barrier()

        pltpu.sync_copy(acc_sh.at[pl.ds(s * RPT, RPT)],
                        out_hbm.at[c].at[pl.ds(s * RPT, RPT)])

    return sc_agg


_sc_agg = _make_sc_agg()


_EPW = NCH * CHUNK


def _pack_body(e_ref, o_ref):
    w = pl.program_id(0)
    e = e_ref[...]
    src = e[0:1, :]
    dst = e[1:2, :]
    pos = w * _EPW + jax.lax.broadcasted_iota(jnp.int32, (1, _EPW), 1)
    p = jnp.where(pos < E,
                  lax.shift_left(src, PBITS) | dst,
                  jnp.int32(N))
    o_ref[...] = p.reshape(1, NCH, CHUNK)


_pack = pl.pallas_call(
    _pack_body,
    grid=(NS,),
    in_specs=[pl.BlockSpec((2, _EPW), lambda w: (0, w))],
    out_specs=pl.BlockSpec((1, NCH, CHUNK), lambda w: (w, 0, 0)),
    out_shape=jax.ShapeDtypeStruct((NS, NCH, CHUNK), jnp.int32),
)


def _epilogue_body(p_ref, deg_ref, w_ref, b_ref, o_ref):
    ssum = jnp.concatenate([p_ref[0], p_ref[1]], axis=1)
    ssum = ssum / deg_ref[...]
    o_ref[...] = lax.dot_general(
        ssum, w_ref[...], (((1,), (1,)), ((), ())),
        preferred_element_type=jnp.float32) + b_ref[...]


_BM = 2000

_epilogue = pl.pallas_call(
    _epilogue_body,
    grid=(N // _BM,),
    in_specs=[
        pl.BlockSpec((NC, _BM, DH), lambda i: (0, i, 0)),
        pl.BlockSpec((_BM, 1), lambda i: (i, 0)),
        pl.BlockSpec((D, D), lambda i: (0, 0)),
        pl.BlockSpec((1, D), lambda i: (0, 0)),
    ],
    out_specs=pl.BlockSpec((_BM, D), lambda i: (i, 0)),
    out_shape=jax.ShapeDtypeStruct((N, D), jnp.float32),
)


def kernel(x, edge_index, node_degrees, W, b):
    ids = _pack(edge_index)

    parts = _sc_agg(x, ids)

    return _epilogue(parts, node_degrees.reshape(N, 1), W, b.reshape(1, D))


# speedup vs baseline: 1.4549x; 1.0664x over previous
import functools

import jax
import jax.numpy as jnp
from jax import lax
from jax.experimental import pallas as pl
from jax.experimental.pallas import tpu as pltpu
from jax.experimental.pallas import tpu_sc as plsc

N = 10000
E = 320000
D = 128
DH = D // 2

NC = 2
NS = 16
CHUNK = 128
NBUF = 3
KAH = 1
NCH = -(-E // (NS * CHUNK * NBUF)) * NBUF
E_PAD = NCH * CHUNK * NS

N_PAD = 10240
RPT = N_PAD // NS
RPX = N // NS
PBITS = 14
PMASK = (1 << PBITS) - 1


def _make_sc_agg():
    mesh = plsc.VectorSubcoreMesh(core_axis_name="c", subcore_axis_name="s")

    @functools.partial(
        pl.kernel,
        mesh=mesh,
        out_type=jax.ShapeDtypeStruct((N_PAD, D), jnp.float32),
        scratch_types=[
            pltpu.VMEM((NCH, CHUNK), jnp.int32),
            pltpu.VMEM((NBUF, 1, CHUNK), jnp.int32),
            pltpu.VMEM((NBUF, 2, CHUNK // 2), jnp.int32),
            pltpu.VMEM((NBUF, CHUNK, DH), jnp.float32),
            pltpu.VMEM_SHARED((N, DH), jnp.float32),
            pltpu.VMEM_SHARED((N_PAD, DH), jnp.float32),
            [pltpu.SemaphoreType.DMA] * NBUF,
            [pltpu.SemaphoreType.DMA] * (2 * NBUF),
        ],
        compiler_params=pltpu.CompilerParams(use_tc_tiling_on_sc=False),
    )
    def sc_agg(x_hbm, ids_hbm, out_hbm,
               pk_v, unp, unpd, bufs, x_sh, acc_sh, gsems, ssems):
        c = lax.axis_index("c")
        s = lax.axis_index("s")

        def zbody(i, carry):
            r = i // (DH // 16)
            k = i - r * (DH // 16)
            bufs[0, r, pl.ds(k * 16, 16)] = jnp.zeros((16,), jnp.float32)
            return carry

        lax.fori_loop(0, CHUNK * (DH // 16), zbody, 0)
        for r5 in range(RPT // CHUNK):
            pltpu.sync_copy(bufs.at[0],
                            acc_sh.at[pl.ds(s * RPT + r5 * CHUNK, CHUNK)])
        pltpu.sync_copy(x_hbm.at[pl.ds(s * RPX, RPX), pl.ds(c * DH, DH)],
                        x_sh.at[pl.ds(s * RPX, RPX)])
        pltpu.sync_copy(ids_hbm.at[s], pk_v)
        plsc.subcore_barrier()

        def unpack(j, b):
            jc = jnp.minimum(jnp.int32(j), jnp.int32(NCH - 1))
            for k in range(CHUNK // 16):
                p = pk_v[jc, pl.ds(k * 16, 16)]
                unp[b, 0, pl.ds(k * 16, 16)] = lax.shift_right_logical(
                    p, PBITS)
                h, kk = divmod(k, CHUNK // 32)
                unpd[b, h, pl.ds(kk * 16, 16)] = lax.bitwise_and(
                    p, jnp.int32(PMASK))

        def gather(j, b):
            pltpu.async_copy(x_sh.at[unp.at[b, 0]], bufs.at[b], gsems[b])

        def gather_wait(j, b):
            pltpu.make_async_copy(x_sh.at[unp.at[b, 0]], bufs.at[b],
                                  gsems[b]).wait()

        def scatter(j, b):
            for h in range(2):
                pltpu.async_copy(bufs.at[b, pl.ds(h * (CHUNK // 2), CHUNK // 2)],
                                 acc_sh.at[unpd.at[b, h]],
                                 ssems[2 * b + h], add=True)

        def scatter_wait(j, b):
            for h in range(2):
                pltpu.make_async_copy(
                    bufs.at[b, pl.ds(h * (CHUNK // 2), CHUNK // 2)],
                    acc_sh.at[unpd.at[b, h]],
                    ssems[2 * b + h]).wait()

        for j in range(KAH):
            unpack(j, j)
            gather(j, j)
        for j in range(NBUF - KAH):
            b, bg = j % NBUF, (j + KAH) % NBUF
            unpack(j + KAH, bg)
            gather(j + KAH, bg)
            gather_wait(j, b)
            scatter(j, b)
        for j in range(NBUF - KAH, NBUF):
            b, bg = j % NBUF, (j + KAH) % NBUF
            scatter_wait(j + KAH - NBUF, bg)
            unpack(j + KAH, bg)
            gather(j + KAH, bg)
            gather_wait(j, b)
            scatter(j, b)

        def body(t, carry):
            for r in range(NBUF):
                j = t * NBUF + r
                bg = (r + KAH) % NBUF
                scatter_wait(j + KAH - NBUF, bg)
                unpack(j + KAH, bg)
                gather(j + KAH, bg)
                gather_wait(j, r)
                scatter(j, r)
            return carry

        lax.fori_loop(1, NCH // NBUF, body, 0)
        for j in range(NCH, NCH + KAH):
            gather_wait(j, j % NBUF)
        for j in range(NCH + KAH - NBUF, NCH):
            scatter_wait(j, j % NBUF)
        plsc.subcore_barrier()

        pltpu.sync_copy(acc_sh.at[pl.ds(s * RPT, RPT)],
                        out_hbm.at[pl.ds(s * RPT, RPT), pl.ds(c * DH, DH)])

    return sc_agg


_sc_agg = _make_sc_agg()


_EPW = NCH * CHUNK


def _pack_body(e_ref, o_ref):
    w = pl.program_id(0)
    e = e_ref[...]
    src = e[0:1, :]
    dst = e[1:2, :]
    pos = w * _EPW + jax.lax.broadcasted_iota(jnp.int32, (1, _EPW), 1)
    p = jnp.where(pos < E,
                  lax.shift_left(src, PBITS) | dst,
                  jnp.int32(N))
    o_ref[...] = p.reshape(1, NCH, CHUNK)


_pack = pl.pallas_call(
    _pack_body,
    grid=(NS,),
    in_specs=[pl.BlockSpec((2, _EPW), lambda w: (0, w))],
    out_specs=pl.BlockSpec((1, NCH, CHUNK), lambda w: (w, 0, 0)),
    out_shape=jax.ShapeDtypeStruct((NS, NCH, CHUNK), jnp.int32),
)


def _epilogue_body(p_ref, deg_ref, w_ref, b_ref, o_ref):
    ssum = p_ref[...] / deg_ref[...]
    o_ref[...] = lax.dot_general(
        ssum, w_ref[...], (((1,), (1,)), ((), ())),
        preferred_element_type=jnp.float32) + b_ref[...]


_BM = 2000

_epilogue = pl.pallas_call(
    _epilogue_body,
    grid=(N // _BM,),
    in_specs=[
        pl.BlockSpec((_BM, D), lambda i: (i, 0)),
        pl.BlockSpec((_BM, 1), lambda i: (i, 0)),
        pl.BlockSpec((D, D), lambda i: (0, 0)),
        pl.BlockSpec((1, D), lambda i: (0, 0)),
    ],
    out_specs=pl.BlockSpec((_BM, D), lambda i: (i, 0)),
    out_shape=jax.ShapeDtypeStruct((N, D), jnp.float32),
)


def kernel(x, edge_index, node_degrees, W, b):
    ids = _pack(edge_index)

    parts = _sc_agg(x, ids)

    return _epilogue(parts, node_degrees.reshape(N, 1), W, b.reshape(1, D))
